# Initial kernel scaffold; baseline (speedup 1.0000x reference)
import jax
import jax.numpy as jnp
from jax.experimental import pallas as pl


def kernel(context, emb, enc_w, enc_b, dec_w, dec_b):
    raise NotImplementedError("write your pallas kernel here")


# speedup vs baseline: 2.6587x; 2.6587x over previous
import functools

import jax
import jax.numpy as jnp
from jax import lax
from jax.experimental import pallas as pl
from jax.experimental.pallas import tpu as pltpu
from jax.experimental.pallas import tpu_sc as plsc

NUM_CORES = 2
NUM_SUBCORES = 16
NW = NUM_CORES * NUM_SUBCORES
IDX_LANES = 128
GROUP = 4
ROWS_PER_GROUP = IDX_LANES * GROUP


def _sc_gather(table, idx2d, n_rows, d):
    per_w = n_rows // NW
    idx_rows = per_w // IDX_LANES
    groups = per_w // ROWS_PER_GROUP
    mesh = plsc.VectorSubcoreMesh(core_axis_name="c", subcore_axis_name="s")

    @functools.partial(
        pl.kernel,
        mesh=mesh,
        out_type=jax.ShapeDtypeStruct((n_rows, d), table.dtype),
        scratch_types=[
            pltpu.VMEM((idx_rows, IDX_LANES), jnp.int32),
            pltpu.VMEM((ROWS_PER_GROUP, d), table.dtype),
            pltpu.SemaphoreType.DMA,
        ],
    )
    def gather_kernel(table_hbm, idx_hbm, out_hbm, idx_v, rows_v, sem):
        wid = lax.axis_index("s") * NUM_CORES + lax.axis_index("c")
        row0 = wid * per_w
        pltpu.sync_copy(idx_hbm.at[pl.ds(wid * idx_rows, idx_rows)], idx_v)

        def body(g, carry):
            copies = [
                pltpu.make_async_copy(
                    table_hbm.at[idx_v.at[g * GROUP + b]],
                    rows_v.at[pl.ds(b * IDX_LANES, IDX_LANES)],
                    sem,
                )
                for b in range(GROUP)
            ]
            for c in copies:
                c.start()
            for c in copies:
                c.wait()
            pltpu.sync_copy(
                rows_v, out_hbm.at[pl.ds(row0 + g * ROWS_PER_GROUP, ROWS_PER_GROUP)])
            return carry

        lax.fori_loop(0, groups, body, 0)

    return gather_kernel(table, idx2d)


def _encoder(flat, enc_w, enc_b2d, bt=512, kc=2560):
    b, k = flat.shape
    e = enc_w.shape[0]
    nb, nk = b // bt, k // kc

    def body(flat_ref, w_ref, b_ref, out_ref, acc_ref):
        kk = pl.program_id(0)
        ii = pl.program_id(1)
        part = lax.dot_general(
            flat_ref[...], w_ref[...], (((1,), (1,)), ((), ())),
            preferred_element_type=jnp.float32)
        sl = pl.ds(ii * bt, bt)

        @pl.when(kk == 0)
        def _():
            acc_ref[sl, :] = part

        @pl.when(kk > 0)
        def _():
            acc_ref[sl, :] = acc_ref[sl, :] + part

        @pl.when(kk == nk - 1)
        def _():
            out_ref[...] = acc_ref[sl, :] + b_ref[...]

    return pl.pallas_call(
        body,
        grid=(nk, nb),
        in_specs=[
            pl.BlockSpec((bt, kc), lambda kk, ii: (ii, kk)),
            pl.BlockSpec((e, kc), lambda kk, ii: (0, kk)),
            pl.BlockSpec((1, e), lambda kk, ii: (0, 0)),
        ],
        out_specs=pl.BlockSpec((bt, e), lambda kk, ii: (ii, 0)),
        out_shape=jax.ShapeDtypeStruct((b, e), jnp.float32),
        scratch_shapes=[pltpu.VMEM((b, e), jnp.float32)],
    )(flat, enc_w, enc_b2d)


def _decoder(encoded, dec_w, dec_b2d, bt=512, nc=2560):
    b, e = encoded.shape
    k = dec_w.shape[0]
    nb, nn = b // bt, k // nc

    def body(enc_ref, w_ref, b_ref, out_ref):
        out_ref[...] = lax.dot_general(
            enc_ref[...], w_ref[...], (((1,), (1,)), ((), ())),
            preferred_element_type=jnp.float32) + b_ref[...]

    return pl.pallas_call(
        body,
        grid=(nn, nb),
        in_specs=[
            pl.BlockSpec((bt, e), lambda nn_, ii: (ii, 0)),
            pl.BlockSpec((nc, e), lambda nn_, ii: (nn_, 0)),
            pl.BlockSpec((1, nc), lambda nn_, ii: (0, nn_)),
        ],
        out_specs=pl.BlockSpec((bt, nc), lambda nn_, ii: (ii, nn_)),
        out_shape=jax.ShapeDtypeStruct((b, k), jnp.float32),
    )(encoded, dec_w, dec_b2d)


def kernel(context, emb, enc_w, enc_b, dec_w, dec_b):
    b, ctx = context.shape
    _, e = emb.shape
    n_rows = b * ctx
    idx2d = context.reshape(n_rows // IDX_LANES, IDX_LANES)
    gathered = _sc_gather(emb, idx2d, n_rows, e)
    flat = gathered.reshape(b, ctx * e)
    encoded = _encoder(flat, enc_w, enc_b.reshape(1, e))
    decoded = _decoder(encoded, dec_w, dec_b.reshape(1, ctx * e))
    return decoded.reshape(b, ctx, e)


# speedup vs baseline: 3.8960x; 1.4653x over previous
import functools

import jax
import jax.numpy as jnp
from jax import lax
from jax.experimental import pallas as pl
from jax.experimental.pallas import tpu as pltpu
from jax.experimental.pallas import tpu_sc as plsc

NUM_CORES = 2
NUM_SUBCORES = 16
NW = NUM_CORES * NUM_SUBCORES
IDX_LANES = 128
GROUP = 4
ROWS_PER_GROUP = IDX_LANES * GROUP


def _sc_gather(table, idx2d, n_rows, d):
    per_w = n_rows // NW
    idx_rows = per_w // IDX_LANES
    groups = per_w // ROWS_PER_GROUP
    mesh = plsc.VectorSubcoreMesh(core_axis_name="c", subcore_axis_name="s")

    @functools.partial(
        pl.kernel,
        mesh=mesh,
        out_type=jax.ShapeDtypeStruct((n_rows, d), table.dtype),
        scratch_types=[
            pltpu.VMEM((idx_rows, IDX_LANES), jnp.int32),
            pltpu.VMEM((ROWS_PER_GROUP, d), table.dtype),
            pltpu.SemaphoreType.DMA,
        ],
    )
    def gather_kernel(table_hbm, idx_hbm, out_hbm, idx_v, rows_v, sem):
        wid = lax.axis_index("s") * NUM_CORES + lax.axis_index("c")
        row0 = wid * per_w
        pltpu.sync_copy(idx_hbm.at[pl.ds(wid * idx_rows, idx_rows)], idx_v)

        def body(g, carry):
            copies = [
                pltpu.make_async_copy(
                    table_hbm.at[idx_v.at[g * GROUP + b]],
                    rows_v.at[pl.ds(b * IDX_LANES, IDX_LANES)],
                    sem,
                )
                for b in range(GROUP)
            ]
            for c in copies:
                c.start()
            for c in copies:
                c.wait()
            pltpu.sync_copy(
                rows_v, out_hbm.at[pl.ds(row0 + g * ROWS_PER_GROUP, ROWS_PER_GROUP)])
            return carry

        lax.fori_loop(0, groups, body, 0)

    return gather_kernel(table, idx2d)


def _encoder(g3, enc_w, enc_b2d, bt=512, tc=40):
    b, ctx, e = g3.shape
    nb, nk = b // bt, ctx // tc

    def body(g_ref, w_ref, b_ref, out_ref, acc_ref):
        kk = pl.program_id(0)
        ii = pl.program_id(1)
        part = lax.dot_general(
            g_ref[:, 0, :], w_ref[:, 0:e], (((1,), (1,)), ((), ())),
            preferred_element_type=jnp.float32)
        for j in range(1, tc):
            part += lax.dot_general(
                g_ref[:, j, :], w_ref[:, j * e:(j + 1) * e],
                (((1,), (1,)), ((), ())),
                preferred_element_type=jnp.float32)
        sl = pl.ds(ii * bt, bt)

        @pl.when(kk == 0)
        def _():
            acc_ref[sl, :] = part

        @pl.when(kk > 0)
        def _():
            acc_ref[sl, :] = acc_ref[sl, :] + part

        @pl.when(kk == nk - 1)
        def _():
            out_ref[...] = acc_ref[sl, :] + b_ref[...]

    return pl.pallas_call(
        body,
        grid=(nk, nb),
        in_specs=[
            pl.BlockSpec((bt, tc, e), lambda kk, ii: (ii, kk, 0)),
            pl.BlockSpec((e, tc * e), lambda kk, ii: (0, kk)),
            pl.BlockSpec((1, e), lambda kk, ii: (0, 0)),
        ],
        out_specs=pl.BlockSpec((bt, e), lambda kk, ii: (ii, 0)),
        out_shape=jax.ShapeDtypeStruct((b, e), jnp.float32),
        scratch_shapes=[pltpu.VMEM((b, e), jnp.float32)],
    )(g3, enc_w, enc_b2d)


def _decoder(encoded, dec_w, dec_b2d, bt=512, tc=40):
    b, e = encoded.shape
    k = dec_w.shape[0]
    ctx = k // e
    nb, nn = b // bt, ctx // tc
    nc = tc * e

    def body(enc_ref, w_ref, b_ref, out_ref):
        enc = enc_ref[...]
        for j in range(tc):
            res = lax.dot_general(
                enc, w_ref[j * e:(j + 1) * e, :], (((1,), (1,)), ((), ())),
                preferred_element_type=jnp.float32)
            out_ref[:, j, :] = res + b_ref[0:1, j * e:(j + 1) * e]

    return pl.pallas_call(
        body,
        grid=(nn, nb),
        in_specs=[
            pl.BlockSpec((bt, e), lambda nn_, ii: (ii, 0)),
            pl.BlockSpec((nc, e), lambda nn_, ii: (nn_, 0)),
            pl.BlockSpec((1, nc), lambda nn_, ii: (0, nn_)),
        ],
        out_specs=pl.BlockSpec((bt, tc, e), lambda nn_, ii: (ii, nn_, 0)),
        out_shape=jax.ShapeDtypeStruct((b, ctx, e), jnp.float32),
    )(encoded, dec_w, dec_b2d)


def kernel(context, emb, enc_w, enc_b, dec_w, dec_b):
    b, ctx = context.shape
    _, e = emb.shape
    n_rows = b * ctx
    idx2d = context.reshape(n_rows // IDX_LANES, IDX_LANES)
    gathered = _sc_gather(emb, idx2d, n_rows, e)
    g3 = gathered.reshape(b, ctx, e)
    encoded = _encoder(g3, enc_w, enc_b.reshape(1, e))
    return _decoder(encoded, dec_w, dec_b.reshape(1, ctx * e))
